# baseline (speedup 1.0000x reference)
import jax
import jax.numpy as jnp
from jax.experimental import pallas as pl
from jax.experimental.pallas import tpu as pltpu

CAP = 65536
X_DIM = 128
Y0, Y1 = 32, 32

GRID = 8
NJ = 4
CH = CAP // GRID
SRCH = CAP // (NJ * GRID)


def _push_body(pos_ref, cnt_ref, srow_ref, crow_ref, sb_in, cb_in, it_in,
               sb_out, cb_out, it_out):
    j = pl.program_id(0)
    i = pl.program_id(1)
    pos = pos_ref[0]
    cnt = cnt_ref[0]

    cbase = i * CH
    clocal = pos - cbase
    cam_in_range = (pos >= cbase) & (pos < cbase + CH)

    @pl.when(cam_in_range)
    def _cam_sel():
        lane = jax.lax.broadcasted_iota(jnp.int32, (Y0 // NJ, Y1, CH), 2)
        crow3 = crow_ref[...][:, :, None]
        cb_out[...] = jnp.where(lane == clocal, crow3, cb_in[...])

    @pl.when(jnp.logical_not(cam_in_range))
    def _cam_copy():
        cb_out[...] = cb_in[...]

    sb_out[...] = sb_in[...]
    it_out[...] = it_in[...]

    sbase = (i * NJ + j) * SRCH
    slocal = pos - sbase

    @pl.when((pos >= sbase) & (pos < sbase + SRCH))
    def _overwrite():
        sb_out[pl.ds(slocal, 1), :] = srow_ref[...]
        col = jax.lax.broadcasted_iota(jnp.int32, (1, 1, SRCH), 2)
        it_out[...] = jnp.where(col == slocal, cnt, it_in[...])


def kernel(state_buffer, cam_data_buffer, iter_buffer, position, state,
           cam_data, count):
    pos2 = position.reshape(1)
    cnt2 = count.reshape(1)
    srow = state.reshape(1, X_DIM)
    crow = cam_data
    cam_t = jax.lax.transpose(cam_data_buffer, (1, 2, 0))
    iter3d = iter_buffer.reshape(NJ * GRID, 1, SRCH)

    out_sb, out_cb, out_it = pl.pallas_call(
        _push_body,
        grid=(NJ, GRID),
        in_specs=[
            pl.BlockSpec(memory_space=pltpu.SMEM),
            pl.BlockSpec(memory_space=pltpu.SMEM),
            pl.BlockSpec((1, X_DIM), lambda j, i: (0, 0)),
            pl.BlockSpec((Y0 // NJ, Y1), lambda j, i: (j, 0)),
            pl.BlockSpec((SRCH, X_DIM), lambda j, i: (i * NJ + j, 0)),
            pl.BlockSpec((Y0 // NJ, Y1, CH), lambda j, i: (j, 0, i)),
            pl.BlockSpec((1, 1, SRCH), lambda j, i: (i * NJ + j, 0, 0)),
        ],
        out_specs=[
            pl.BlockSpec((SRCH, X_DIM), lambda j, i: (i * NJ + j, 0)),
            pl.BlockSpec((Y0 // NJ, Y1, CH), lambda j, i: (j, 0, i)),
            pl.BlockSpec((1, 1, SRCH), lambda j, i: (i * NJ + j, 0, 0)),
        ],
        out_shape=[
            jax.ShapeDtypeStruct((CAP, X_DIM), jnp.float32),
            jax.ShapeDtypeStruct((Y0, Y1, CAP), jnp.float32),
            jax.ShapeDtypeStruct((NJ * GRID, 1, SRCH), jnp.int32),
        ],
        compiler_params=pltpu.CompilerParams(
            dimension_semantics=("arbitrary", "arbitrary"),
        ),
    )(pos2, cnt2, srow, crow, state_buffer, cam_t, iter3d)

    new_position = jnp.remainder(position + 1, CAP)
    full_buffer = (position + 1) == CAP
    return (out_sb, jax.lax.transpose(out_cb, (2, 0, 1)),
            out_it.reshape(CAP), new_position, full_buffer)
